# baseline (speedup 1.0000x reference)
import jax
import jax.numpy as jnp
from jax import lax
from jax.experimental import pallas as pl
from jax.experimental.pallas import tpu as pltpu

_B = 4
_N = 2048
_NFEAT = 128
_NHID = 64
_NHEADS = 4
_OUTC = 8
_ROWS = 256
_SLOPE = 0.2

_NT = (((1,), (1,)), ((), ()))
_HI = lax.Precision.HIGHEST


def _leaky(v):
    return jnp.where(v >= 0, v, _SLOPE * v)


def _elu(v):
    return jnp.where(v > 0, v, jnp.exp(jnp.minimum(v, 0.0)) - 1.0)


def _gat_body(x_ref, wh_ref, ah_ref, wo_ref, ao_ref, wl_ref, out_ref, h_scr, s_scr):
    f32 = jnp.float32
    xb = x_ref[0]

    for k in range(_NHEADS):
        Wk = wh_ref[k]
        Wh = jnp.dot(xb, Wk, preferred_element_type=f32, precision=_HI)
        a1 = ah_ref[k:k + 1, 0:_NHID]
        a2 = ah_ref[k:k + 1, _NHID:]
        s_scr[:, :] = lax.dot_general(Wh, a1, _NT, preferred_element_type=f32, precision=_HI)
        tT = lax.dot_general(a2, Wh, _NT, preferred_element_type=f32, precision=_HI)
        tmax = jnp.max(tT)

        def l1_chunk(c, _, Wh=Wh, tT=tT, tmax=tmax, k=k):
            sc = s_scr[pl.ds(c * _ROWS, _ROWS), :]
            m = _leaky(sc + tmax)
            p = jnp.exp(_leaky(sc + tT) - m)
            z = jnp.sum(p, axis=1, keepdims=True)
            hc = jnp.dot(p, Wh, preferred_element_type=f32, precision=_HI) / z
            h_scr[pl.ds(c * _ROWS, _ROWS), k * _NHID:(k + 1) * _NHID] = _elu(hc)
            return 0

        lax.fori_loop(0, _N // _ROWS, l1_chunk, 0)

    h = h_scr[:, :]
    Wh2 = jnp.dot(h, wo_ref[:, :], preferred_element_type=f32, precision=_HI)
    a1o = ao_ref[0:1, 0:_OUTC]
    a2o = ao_ref[0:1, _OUTC:]
    s_scr[:, :] = lax.dot_general(Wh2, a1o, _NT, preferred_element_type=f32, precision=_HI)
    t2T = lax.dot_general(a2o, Wh2, _NT, preferred_element_type=f32, precision=_HI)
    tmax2 = jnp.max(t2T)

    def l2_chunk(c, acc):
        sc = s_scr[pl.ds(c * _ROWS, _ROWS), :]
        m = _leaky(sc + tmax2)
        p = jnp.exp(_leaky(sc + t2T) - m)
        z = jnp.sum(p, axis=1, keepdims=True)
        hc = jnp.dot(p, Wh2, preferred_element_type=f32, precision=_HI) / z
        wc = wl_ref[pl.ds(c * _ROWS, _ROWS), :]
        return acc + jnp.sum(_elu(hc) * wc)

    acc = lax.fori_loop(0, _N // _ROWS, l2_chunk, jnp.float32(0.0))
    out_ref[0] = jnp.full((8, 128), acc, dtype=f32)


def kernel(x, W_heads, a_heads, W_out, a_out, W_lin, b_lin):
    ah = a_heads.reshape(_NHEADS, 2 * _NHID)
    ao = a_out.reshape(1, 2 * _OUTC)
    wl = W_lin.reshape(_N, _OUTC)
    out = pl.pallas_call(
        _gat_body,
        grid=(_B,),
        in_specs=[
            pl.BlockSpec((1, _N, _NFEAT), lambda b: (b, 0, 0)),
            pl.BlockSpec((_NHEADS, _NFEAT, _NHID), lambda b: (0, 0, 0)),
            pl.BlockSpec((_NHEADS, 2 * _NHID), lambda b: (0, 0)),
            pl.BlockSpec((_NHEADS * _NHID, _OUTC), lambda b: (0, 0)),
            pl.BlockSpec((1, 2 * _OUTC), lambda b: (0, 0)),
            pl.BlockSpec((_N, _OUTC), lambda b: (0, 0)),
        ],
        out_specs=pl.BlockSpec((1, 8, 128), lambda b: (b, 0, 0)),
        out_shape=jax.ShapeDtypeStruct((_B, 8, 128), jnp.float32),
        scratch_shapes=[pltpu.VMEM((_N, _NHEADS * _NHID), jnp.float32),
                        pltpu.VMEM((_N, 1), jnp.float32)],
    )(x, W_heads, ah, W_out, ao, wl)
    return out[:, 0, :1] + b_lin


# speedup vs baseline: 1.9154x; 1.9154x over previous
import jax
import jax.numpy as jnp
from jax import lax
from jax.experimental import pallas as pl
from jax.experimental.pallas import tpu as pltpu

_B = 4
_N = 2048
_NFEAT = 128
_NHID = 64
_NHEADS = 4
_OUTC = 8
_ROWS = 256
_SLOPE = 0.2

_NT = (((1,), (1,)), ((), ()))
_HI = lax.Precision.HIGHEST
_BF = jnp.bfloat16


def _leaky(v):
    return jnp.where(v >= 0, v, _SLOPE * v)


def _elu(v):
    return jnp.where(v > 0, v, jnp.exp(jnp.minimum(v, 0.0)) - 1.0)


def _hilo(v):
    hi = v.astype(_BF)
    lo = (v - hi.astype(jnp.float32)).astype(_BF)
    return hi, lo


def _att_tables(Wh, s_col, t_col):
    f32 = jnp.float32
    tmax = jnp.max(t_col)
    w1 = jnp.exp(t_col - tmax)
    w2 = jnp.exp(_SLOPE * (t_col - tmax))
    r1h, r1l = _hilo(w1 * Wh)
    r2h, r2l = _hilo(w2 * Wh)
    zc = jnp.concatenate([w1, w2], axis=1)
    zch, zcl = _hilo(zc)
    rhs = jnp.concatenate([r1h, r1l, r2h, r2l, zch, zcl], axis=1)
    t2row = jnp.sum(w2 * Wh, axis=0, keepdims=True)
    tot2 = jnp.sum(w2)
    return rhs, t2row, tot2, tmax


def _att_chunk(sc, tT, rhs, t2row, tot2, tmax, width):
    f32 = jnp.float32
    mb = ((sc + tT) > 0).astype(_BF)
    p = jnp.dot(mb, rhs, preferred_element_type=f32)
    w = width
    A = p[:, 0:w] + p[:, w:2 * w]
    Bt = p[:, 2 * w:3 * w] + p[:, 3 * w:4 * w]
    z1 = p[:, 4 * w:4 * w + 1] + p[:, 4 * w + 2:4 * w + 3]
    z2t = p[:, 4 * w + 1:4 * w + 2] + p[:, 4 * w + 3:4 * w + 4]
    spt = sc + tmax
    m = _leaky(spt)
    cA = jnp.exp(spt - m)
    cB = jnp.exp(_SLOPE * spt - m)
    num = cA * A + cB * (t2row - Bt)
    den = cA * z1 + cB * (tot2 - z2t)
    return num / den


def _gat_body(x_ref, wh_ref, ah_ref, wo_ref, ao_ref, wl_ref, out_ref, h_scr, s_scr):
    f32 = jnp.float32
    xb = x_ref[0]

    for k in range(_NHEADS):
        Wk = wh_ref[k]
        Wh = jnp.dot(xb, Wk, preferred_element_type=f32, precision=_HI)
        a1 = ah_ref[k:k + 1, 0:_NHID]
        a2 = ah_ref[k:k + 1, _NHID:]
        s_col = lax.dot_general(Wh, a1, _NT, preferred_element_type=f32, precision=_HI)
        t_col = lax.dot_general(Wh, a2, _NT, preferred_element_type=f32, precision=_HI)
        tT = lax.dot_general(a2, Wh, _NT, preferred_element_type=f32, precision=_HI)
        s_scr[:, :] = s_col
        rhs, t2row, tot2, tmax = _att_tables(Wh, s_col, t_col)

        def l1_chunk(c, _, tT=tT, rhs=rhs, t2row=t2row, tot2=tot2, tmax=tmax, k=k):
            sc = s_scr[pl.ds(c * _ROWS, _ROWS), :]
            hc = _att_chunk(sc, tT, rhs, t2row, tot2, tmax, _NHID)
            h_scr[pl.ds(c * _ROWS, _ROWS), k * _NHID:(k + 1) * _NHID] = _elu(hc)
            return 0

        lax.fori_loop(0, _N // _ROWS, l1_chunk, 0)

    h = h_scr[:, :]
    Wh2 = jnp.dot(h, wo_ref[:, :], preferred_element_type=f32, precision=_HI)
    a1o = ao_ref[0:1, 0:_OUTC]
    a2o = ao_ref[0:1, _OUTC:]
    s2_col = lax.dot_general(Wh2, a1o, _NT, preferred_element_type=f32, precision=_HI)
    t2_col = lax.dot_general(Wh2, a2o, _NT, preferred_element_type=f32, precision=_HI)
    t2T = lax.dot_general(a2o, Wh2, _NT, preferred_element_type=f32, precision=_HI)
    s_scr[:, :] = s2_col
    rhs2, t2row2, tot22, tmax2 = _att_tables(Wh2, s2_col, t2_col)

    def l2_chunk(c, acc):
        sc = s_scr[pl.ds(c * _ROWS, _ROWS), :]
        hc = _att_chunk(sc, t2T, rhs2, t2row2, tot22, tmax2, _OUTC)
        wc = wl_ref[pl.ds(c * _ROWS, _ROWS), :]
        return acc + jnp.sum(_elu(hc) * wc)

    acc = lax.fori_loop(0, _N // _ROWS, l2_chunk, jnp.float32(0.0))
    out_ref[0] = jnp.full((8, 128), acc, dtype=f32)


def kernel(x, W_heads, a_heads, W_out, a_out, W_lin, b_lin):
    ah = a_heads.reshape(_NHEADS, 2 * _NHID)
    ao = a_out.reshape(1, 2 * _OUTC)
    wl = W_lin.reshape(_N, _OUTC)
    out = pl.pallas_call(
        _gat_body,
        grid=(_B,),
        in_specs=[
            pl.BlockSpec((1, _N, _NFEAT), lambda b: (b, 0, 0)),
            pl.BlockSpec((_NHEADS, _NFEAT, _NHID), lambda b: (0, 0, 0)),
            pl.BlockSpec((_NHEADS, 2 * _NHID), lambda b: (0, 0)),
            pl.BlockSpec((_NHEADS * _NHID, _OUTC), lambda b: (0, 0)),
            pl.BlockSpec((1, 2 * _OUTC), lambda b: (0, 0)),
            pl.BlockSpec((_N, _OUTC), lambda b: (0, 0)),
        ],
        out_specs=pl.BlockSpec((1, 8, 128), lambda b: (b, 0, 0)),
        out_shape=jax.ShapeDtypeStruct((_B, 8, 128), jnp.float32),
        scratch_shapes=[pltpu.VMEM((_N, _NHEADS * _NHID), jnp.float32),
                        pltpu.VMEM((_N, 1), jnp.float32)],
    )(x, W_heads, ah, W_out, ao, wl)
    return out[:, 0, :1] + b_lin


# speedup vs baseline: 2.5066x; 1.3086x over previous
import jax
import jax.numpy as jnp
from jax import lax
from jax.experimental import pallas as pl
from jax.experimental.pallas import tpu as pltpu

_B = 4
_N = 2048
_NFEAT = 128
_NHID = 64
_NHEADS = 4
_OUTC = 8
_ROWS = 512
_SLOPE = 0.2

_NT = (((1,), (1,)), ((), ()))
_HI = lax.Precision.HIGHEST
_BF = jnp.bfloat16


def _leaky(v):
    return jnp.where(v >= 0, v, _SLOPE * v)


def _elu(v):
    return jnp.where(v > 0, v, jnp.exp(jnp.minimum(v, 0.0)) - 1.0)


def _hilo(v):
    hi = v.astype(_BF)
    lo = (v - hi.astype(jnp.float32)).astype(_BF)
    return hi, lo


def _att_tables(Wh, s_col, t_col):
    f32 = jnp.float32
    tmax = jnp.max(t_col)
    w1 = jnp.exp(t_col - tmax)
    w2 = jnp.exp(_SLOPE * (t_col - tmax))
    r1h, r1l = _hilo(w1 * Wh)
    r2h, r2l = _hilo(w2 * Wh)
    zc = jnp.concatenate([w1, w2], axis=1)
    zch, zcl = _hilo(zc)
    rhs = jnp.concatenate([r1h, r1l, r2h, r2l, zch, zcl], axis=1)
    t2row = jnp.sum(w2 * Wh, axis=0, keepdims=True)
    tot2 = jnp.sum(w2)
    return rhs, t2row, tot2, tmax


def _att_chunk(sc, tT, rhs, t2row, tot2, tmax, width):
    f32 = jnp.float32
    mb = ((sc + tT) > 0).astype(_BF)
    p = jnp.dot(mb, rhs, preferred_element_type=f32)
    w = width
    A = p[:, 0:w] + p[:, w:2 * w]
    Bt = p[:, 2 * w:3 * w] + p[:, 3 * w:4 * w]
    z1 = p[:, 4 * w:4 * w + 1] + p[:, 4 * w + 2:4 * w + 3]
    z2t = p[:, 4 * w + 1:4 * w + 2] + p[:, 4 * w + 3:4 * w + 4]
    spt = sc + tmax
    m = _leaky(spt)
    cA = jnp.exp(spt - m)
    cB = jnp.exp(_SLOPE * spt - m)
    num = cA * A + cB * (t2row - Bt)
    den = cA * z1 + cB * (tot2 - z2t)
    return num / den


def _gat_body(x_ref, wh_ref, ah_ref, wo_ref, ao_ref, wl_ref, out_ref, h_scr, s_scr):
    f32 = jnp.float32
    xb = x_ref[0]

    for k in range(_NHEADS):
        Wk = wh_ref[k]
        Wh = jnp.dot(xb, Wk, preferred_element_type=f32, precision=_HI)
        a1 = ah_ref[k:k + 1, 0:_NHID]
        a2 = ah_ref[k:k + 1, _NHID:]
        s_col = jnp.sum(Wh * a1, axis=1, keepdims=True)
        t_col = jnp.sum(Wh * a2, axis=1, keepdims=True)
        tT = lax.dot_general(a2, Wh, _NT, preferred_element_type=f32)
        s_scr[:, :] = s_col
        rhs, t2row, tot2, tmax = _att_tables(Wh, s_col, t_col)

        def l1_chunk(c, _, tT=tT, rhs=rhs, t2row=t2row, tot2=tot2, tmax=tmax, k=k):
            sc = s_scr[pl.ds(c * _ROWS, _ROWS), :]
            hc = _att_chunk(sc, tT, rhs, t2row, tot2, tmax, _NHID)
            h_scr[pl.ds(c * _ROWS, _ROWS), k * _NHID:(k + 1) * _NHID] = _elu(hc)
            return 0

        lax.fori_loop(0, _N // _ROWS, l1_chunk, 0)

    h = h_scr[:, :]
    Wh2 = jnp.dot(h, wo_ref[:, :], preferred_element_type=f32, precision=_HI)
    a1o = ao_ref[0:1, 0:_OUTC]
    a2o = ao_ref[0:1, _OUTC:]
    s2_col = jnp.sum(Wh2 * a1o, axis=1, keepdims=True)
    t2_col = jnp.sum(Wh2 * a2o, axis=1, keepdims=True)
    t2T = lax.dot_general(a2o, Wh2, _NT, preferred_element_type=f32)
    s_scr[:, :] = s2_col
    rhs2, t2row2, tot22, tmax2 = _att_tables(Wh2, s2_col, t2_col)

    def l2_chunk(c, acc):
        sc = s_scr[pl.ds(c * _ROWS, _ROWS), :]
        hc = _att_chunk(sc, t2T, rhs2, t2row2, tot22, tmax2, _OUTC)
        wc = wl_ref[pl.ds(c * _ROWS, _ROWS), :]
        return acc + jnp.sum(_elu(hc) * wc)

    acc = lax.fori_loop(0, _N // _ROWS, l2_chunk, jnp.float32(0.0))
    out_ref[0] = jnp.full((8, 128), acc, dtype=f32)


def kernel(x, W_heads, a_heads, W_out, a_out, W_lin, b_lin):
    ah = a_heads.reshape(_NHEADS, 2 * _NHID)
    ao = a_out.reshape(1, 2 * _OUTC)
    wl = W_lin.reshape(_N, _OUTC)
    out = pl.pallas_call(
        _gat_body,
        grid=(_B,),
        in_specs=[
            pl.BlockSpec((1, _N, _NFEAT), lambda b: (b, 0, 0)),
            pl.BlockSpec((_NHEADS, _NFEAT, _NHID), lambda b: (0, 0, 0)),
            pl.BlockSpec((_NHEADS, 2 * _NHID), lambda b: (0, 0)),
            pl.BlockSpec((_NHEADS * _NHID, _OUTC), lambda b: (0, 0)),
            pl.BlockSpec((1, 2 * _OUTC), lambda b: (0, 0)),
            pl.BlockSpec((_N, _OUTC), lambda b: (0, 0)),
        ],
        out_specs=pl.BlockSpec((1, 8, 128), lambda b: (b, 0, 0)),
        out_shape=jax.ShapeDtypeStruct((_B, 8, 128), jnp.float32),
        scratch_shapes=[pltpu.VMEM((_N, _NHEADS * _NHID), jnp.float32),
                        pltpu.VMEM((_N, 1), jnp.float32)],
    )(x, W_heads, ah, W_out, ao, wl)
    return out[:, 0, :1] + b_lin
